# Initial kernel scaffold; baseline (speedup 1.0000x reference)
import jax
import jax.numpy as jnp
from jax.experimental import pallas as pl


def kernel(inputs, targets, alpha):
    raise NotImplementedError("write your pallas kernel here")


# speedup vs baseline: 3.1588x; 3.1588x over previous
import jax
import jax.numpy as jnp
from jax import lax
from jax.experimental import pallas as pl


def _focal_body(x_ref, t_ref, a_ref, o_ref, *, n_total):
    i = pl.program_id(0)
    x = x_ref[...]
    t = t_ref[...]
    r, c = x.shape
    m = jnp.max(x, axis=1, keepdims=True)
    z = jnp.sum(jnp.exp(x - m), axis=1, keepdims=True)
    col = lax.broadcasted_iota(jnp.int32, (r, c), 1)
    msk = col == t
    xt = jnp.sum(jnp.where(msk, x, 0.0), axis=1, keepdims=True)
    at = jnp.sum(jnp.where(msk, a_ref[...], 0.0), axis=1, keepdims=True)
    logp = (xt - m) - jnp.log(z)
    p = jnp.exp(logp)
    q = 1.0 - p
    loss = -at * q * q * logp
    s = jnp.sum(loss, keepdims=True).reshape(1, 1) * (1.0 / n_total)

    @pl.when(i == 0)
    def _():
        o_ref[...] = jnp.zeros_like(o_ref)

    o_ref[...] += s


def kernel(inputs, targets, alpha):
    n, c = inputs.shape
    R = 512
    nb = n // R
    t2 = targets.reshape(n, 1)
    a_row = alpha.reshape(1, c)

    import functools
    out = pl.pallas_call(
        functools.partial(_focal_body, n_total=n),
        grid=(nb,),
        in_specs=[
            pl.BlockSpec((R, c), lambda i: (i, 0)),
            pl.BlockSpec((R, 1), lambda i: (i, 0)),
            pl.BlockSpec((1, c), lambda i: (0, 0)),
        ],
        out_specs=pl.BlockSpec((1, 1), lambda i: (0, 0)),
        out_shape=jax.ShapeDtypeStruct((1, 1), jnp.float32),
    )(inputs, t2, a_row)
    return out[0, 0]


# speedup vs baseline: 3.4523x; 1.0929x over previous
import jax
import jax.numpy as jnp
from jax import lax
from jax.experimental import pallas as pl


def _focal_body(x_ref, t_ref, a_ref, o_ref, *, n_total):
    i = pl.program_id(0)
    x = x_ref[...]
    t = t_ref[...]
    r, c = x.shape
    m = jnp.max(x, axis=1, keepdims=True)
    z = jnp.sum(jnp.exp(x - m), axis=1, keepdims=True)
    col = lax.broadcasted_iota(jnp.int32, (r, c), 1)
    msk = col == t
    xt = jnp.sum(jnp.where(msk, x, 0.0), axis=1, keepdims=True)
    at = jnp.sum(jnp.where(msk, a_ref[...], 0.0), axis=1, keepdims=True)
    logp = (xt - m) - jnp.log(z)
    p = jnp.exp(logp)
    q = 1.0 - p
    loss = -at * q * q * logp
    s = jnp.sum(loss, keepdims=True).reshape(1, 1) * (1.0 / n_total)

    @pl.when(i == 0)
    def _():
        o_ref[...] = jnp.zeros_like(o_ref)

    o_ref[...] += s


def kernel(inputs, targets, alpha):
    n, c = inputs.shape
    R = 1024
    nb = n // R
    t2 = targets.reshape(n, 1)
    a_row = alpha.reshape(1, c)

    import functools
    out = pl.pallas_call(
        functools.partial(_focal_body, n_total=n),
        grid=(nb,),
        in_specs=[
            pl.BlockSpec((R, c), lambda i: (i, 0)),
            pl.BlockSpec((R, 1), lambda i: (i, 0)),
            pl.BlockSpec((1, c), lambda i: (0, 0)),
        ],
        out_specs=pl.BlockSpec((1, 1), lambda i: (0, 0)),
        out_shape=jax.ShapeDtypeStruct((1, 1), jnp.float32),
    )(inputs, t2, a_row)
    return out[0, 0]


# speedup vs baseline: 3.4979x; 1.0132x over previous
import jax
import jax.numpy as jnp
from jax import lax
from jax.experimental import pallas as pl


def _focal_body(x_ref, t_ref, a_ref, o_ref, *, n_total):
    i = pl.program_id(0)
    x = x_ref[...]
    t = t_ref[...]
    r, c = x.shape
    m = jnp.max(x, axis=1, keepdims=True)
    z = jnp.sum(jnp.exp(x - m), axis=1, keepdims=True)
    col = lax.broadcasted_iota(jnp.int32, (r, c), 1)
    msk = col == t
    xt = jnp.sum(jnp.where(msk, x, 0.0), axis=1, keepdims=True)
    at = jnp.sum(jnp.where(msk, a_ref[...], 0.0), axis=1, keepdims=True)
    logp = (xt - m) - jnp.log(z)
    p = jnp.exp(logp)
    q = 1.0 - p
    loss = -at * q * q * logp
    s = jnp.sum(loss, keepdims=True).reshape(1, 1) * (1.0 / n_total)

    @pl.when(i == 0)
    def _():
        o_ref[...] = jnp.zeros_like(o_ref)

    o_ref[...] += s


def kernel(inputs, targets, alpha):
    n, c = inputs.shape
    R = 2048
    nb = n // R
    t2 = targets.reshape(n, 1)
    a_row = alpha.reshape(1, c)

    import functools
    out = pl.pallas_call(
        functools.partial(_focal_body, n_total=n),
        grid=(nb,),
        in_specs=[
            pl.BlockSpec((R, c), lambda i: (i, 0)),
            pl.BlockSpec((R, 1), lambda i: (i, 0)),
            pl.BlockSpec((1, c), lambda i: (0, 0)),
        ],
        out_specs=pl.BlockSpec((1, 1), lambda i: (0, 0)),
        out_shape=jax.ShapeDtypeStruct((1, 1), jnp.float32),
    )(inputs, t2, a_row)
    return out[0, 0]
